# baseline (speedup 1.0000x reference)
import functools

import jax
import jax.numpy as jnp
from jax import lax
from jax.experimental import pallas as pl
from jax.experimental.pallas import tpu as pltpu
from jax.experimental.pallas import tpu_sc as plsc

L = 1024
KNN = 30
KPAD = 32
NB = 4
N = NB * L
NC = 2
NS = 16
NW = NC * NS
E_PER_W = N // NW
HI = jax.lax.Precision.HIGHEST
BF = jnp.bfloat16
F32 = jnp.float32


def _knn_body(xf_ref, W1_ref, b1_ref, g1_ref, be1_ref, rm1_ref, rv1_ref,
              nbr_ref, h1_ref, vals_ref):
    b = pl.program_id(0)
    ft = xf_ref[0]
    sq = jnp.sum(ft * ft, axis=1, keepdims=True)
    sq_row = jnp.reshape(jnp.sum(ft * ft, axis=1), (1, L))
    G = jax.lax.dot_general(ft, ft, (((1,), (1,)), ((), ())))
    vals_ref[...] = sq + sq_row - 2.0 * G

    cols = jax.lax.broadcasted_iota(jnp.int32, (L, L), 1)
    tcols = jax.lax.broadcasted_iota(jnp.int32, (L, KPAD), 1)

    def step(t, nbr):
        vals = vals_ref[...]
        m = jnp.min(vals, axis=1, keepdims=True)
        eq = vals == m
        idxm = jnp.min(jnp.where(eq, cols, L), axis=1, keepdims=True)
        vals_ref[...] = jnp.where(cols == idxm, jnp.inf, vals)
        return jnp.where(tcols == t, idxm, nbr)

    nbr = jax.lax.fori_loop(
        0, KNN, step, jnp.zeros((L, KPAD), jnp.int32), unroll=2)
    nbr_ref[0] = nbr + b * L

    h = jax.lax.dot_general(ft, W1_ref[...], (((1,), (0,)), ((), ())))
    h = (h + b1_ref[0] - rm1_ref[0]) / jnp.sqrt(rv1_ref[0] + 1e-5) \
        * g1_ref[0] + be1_ref[0]
    h1_ref[0] = h


def _knn_call(xf, W1, b1, g1, be1, rm1, rv1):
    vec = lambda v: v.reshape(1, -1)
    full = lambda r: pl.BlockSpec((1, r.shape[1]), lambda i: (0, 0))
    hid = W1.shape[1]
    return pl.pallas_call(
        _knn_body,
        grid=(NB,),
        in_specs=[
            pl.BlockSpec((1, L, xf.shape[2]), lambda i: (i, 0, 0)),
            pl.BlockSpec(W1.shape, lambda i: (0, 0)),
            full(vec(b1)), full(vec(g1)), full(vec(be1)),
            full(vec(rm1)), full(vec(rv1)),
        ],
        out_specs=[
            pl.BlockSpec((1, L, KPAD), lambda i: (i, 0, 0)),
            pl.BlockSpec((1, L, hid), lambda i: (i, 0, 0)),
        ],
        out_shape=[
            jax.ShapeDtypeStruct((NB, L, KPAD), jnp.int32),
            jax.ShapeDtypeStruct((NB, L, hid), jnp.float32),
        ],
        scratch_shapes=[pltpu.VMEM((L, L), jnp.float32)],
    )(xf, W1, vec(b1), vec(g1), vec(be1), vec(rm1), vec(rv1))


def _v2v_sc(d, with_deg):
    nj = d // 16
    mesh = plsc.VectorSubcoreMesh(core_axis_name="c", subcore_axis_name="s")
    rows_per_tile = N // NS

    out_type = [jax.ShapeDtypeStruct((NC, N, d), jnp.float32)]
    if with_deg:
        out_type.append(jax.ShapeDtypeStruct((NC, N), jnp.float32))

    scratch = [
        pltpu.VMEM((E_PER_W, KPAD), jnp.int32),
        pltpu.VMEM((2, KPAD, d), jnp.float32),
        pltpu.VMEM((2, KPAD, d), jnp.float32),
        pltpu.VMEM((rows_per_tile, d), jnp.float32),
        pltpu.VMEM((KPAD,), jnp.float32),
        pltpu.VMEM((rows_per_tile,), jnp.float32),
        pltpu.VMEM_SHARED((N, d), jnp.float32),
        pltpu.VMEM_SHARED((N,), jnp.float32),
        pltpu.SemaphoreType.DMA,
    ]

    def body(gnbr_hbm, h_hbm, vsum_out, *rest):
        if with_deg:
            deg_out = rest[0]
            rest = rest[1:]
        (idxs, rows, rep, zbuf, ones_v, zdeg, vsum_sh, deg_sh, gsem) = rest
        cid = lax.axis_index("c")
        sid = lax.axis_index("s")
        wid = sid * NC + cid
        base = wid * E_PER_W

        zero16 = jnp.zeros((16,), F32)
        def zrow(i, _):
            for j in range(nj):
                zbuf[i, pl.ds(16 * j, 16)] = zero16
            return 0
        lax.fori_loop(0, rows_per_tile, zrow, 0)
        pltpu.sync_copy(zbuf, vsum_sh.at[pl.ds(sid * rows_per_tile,
                                               rows_per_tile)])
        if with_deg:
            for j in range(rows_per_tile // 16):
                zdeg[pl.ds(16 * j, 16)] = zero16
            pltpu.sync_copy(zdeg, deg_sh.at[pl.ds(sid * rows_per_tile,
                                                  rows_per_tile)])
            lane = lax.iota(jnp.int32, 16)
            ones_v[pl.ds(0, 16)] = jnp.ones((16,), F32)
            ones_v[pl.ds(16, 16)] = jnp.where(lane + 16 < KNN, 1.0, 0.0
                                              ).astype(F32)
        for bb in range(2):
            for r in range(KNN, KPAD):
                for j in range(nj):
                    rep[bb, r, pl.ds(16 * j, 16)] = zero16

        pltpu.sync_copy(gnbr_hbm.at[pl.ds(base, E_PER_W)], idxs)
        plsc.subcore_barrier()

        pltpu.async_copy(h_hbm.at[idxs.at[0]], rows.at[0], gsem)

        def edge(eh, _):
            for bb in range(2):
                le = eh * 2 + bb

                @pl.when(le + 1 < E_PER_W)
                def _():
                    pltpu.async_copy(h_hbm.at[idxs.at[le + 1]],
                                     rows.at[1 - bb], gsem)

                pltpu.make_async_copy(h_hbm.at[idxs.at[le]], rows.at[bb],
                                      gsem).wait()
                for j in range(nj):
                    acc = rows[bb, 0, pl.ds(16 * j, 16)]
                    for r in range(1, KNN):
                        acc = acc + rows[bb, r, pl.ds(16 * j, 16)]
                    acc = acc * (1.0 / KNN)
                    for r in range(KNN):
                        rep[bb, r, pl.ds(16 * j, 16)] = acc
                pltpu.sync_copy(rep.at[bb], vsum_sh.at[idxs.at[le]],
                                add=True)
                if with_deg:
                    pltpu.sync_copy(ones_v, deg_sh.at[idxs.at[le]],
                                    add=True)
            return 0

        lax.fori_loop(0, E_PER_W // 2, edge, 0, unroll=False)

        plsc.subcore_barrier()
        sl = pl.ds(sid * rows_per_tile, rows_per_tile)
        pltpu.sync_copy(vsum_sh.at[sl], vsum_out.at[cid, sl])
        if with_deg:
            pltpu.sync_copy(deg_sh.at[sl], deg_out.at[cid, sl])

    return pl.kernel(body, out_type=out_type, mesh=mesh,
                     scratch_types=scratch,
                     compiler_params=pltpu.CompilerParams(
                         use_tc_tiling_on_sc=False))


def _mid_body(v_ref, dp_ref, W2_ref, b2_ref, g2_ref, be2_ref, rm2_ref,
              rv2_ref, h2_ref, degc_ref):
    Vsum = v_ref[0] + v_ref[1]
    ones2 = jnp.ones((NC, 1), jnp.float32)
    deg = jax.lax.dot_general(dp_ref[...], ones2, (((0,), (0,)), ((), ())),
                              precision=HI)
    degc = jnp.maximum(deg, 1.0)
    degc_ref[...] = degc
    V = jax.nn.relu(Vsum / degc)
    h = jax.lax.dot_general(V, W2_ref[...], (((1,), (0,)), ((), ())))
    h2_ref[...] = (h + b2_ref[0] - rm2_ref[0]) / jnp.sqrt(rv2_ref[0] + 1e-5) \
        * g2_ref[0] + be2_ref[0]


def _mid_call(vp, degp, W2, b2, g2, be2, rm2, rv2):
    vec = lambda v: v.reshape(1, -1)
    nospec = lambda a: pl.BlockSpec(a.shape, lambda: tuple(0 for _ in a.shape))
    out_c = W2.shape[1]
    args = (vp, degp, W2, vec(b2), vec(g2), vec(be2), vec(rm2), vec(rv2))
    return pl.pallas_call(
        _mid_body,
        in_specs=[nospec(a) for a in args],
        out_specs=[
            pl.BlockSpec((N, out_c), lambda: (0, 0)),
            pl.BlockSpec((N, 1), lambda: (0, 0)),
        ],
        out_shape=[
            jax.ShapeDtypeStruct((N, out_c), jnp.float32),
            jax.ShapeDtypeStruct((N, 1), jnp.float32),
        ],
    )(*args)


def _fin_body(v_ref, degc_ref, out_ref):
    out_ref[...] = (v_ref[0] + v_ref[1]) / degc_ref[...]


def _fin_call(vp, degc):
    nospec = lambda a: pl.BlockSpec(a.shape, lambda: tuple(0 for _ in a.shape))
    return pl.pallas_call(
        _fin_body,
        in_specs=[nospec(vp), nospec(degc)],
        out_specs=pl.BlockSpec(vp.shape[1:], lambda: (0, 0)),
        out_shape=jax.ShapeDtypeStruct(vp.shape[1:], jnp.float32),
    )(vp, degc)


def kernel(x, W1, b1, g1, be1, rm1, rv1, W2, b2, g2, be2, rm2, rv2):
    B, C, H, W = x.shape
    hid, out_c = W1.shape[1], W2.shape[1]
    xf = x.reshape(B, L, C)

    nbr, h1 = _knn_call(xf, W1, b1, g1, be1, rm1, rv1)
    gnbr = nbr.reshape(N, KPAD)
    h1f = h1.reshape(N, hid)

    v1p, degp = _v2v_sc(hid, True)(gnbr, h1f)
    h2, degc = _mid_call(v1p, degp, W2, b2, g2, be2, rm2, rv2)
    v2p, = _v2v_sc(out_c, False)(gnbr, h2)
    out = _fin_call(v2p, degc)

    return out.reshape(B, -1, H, W)
